# Initial kernel scaffold; baseline (speedup 1.0000x reference)
import jax
import jax.numpy as jnp
from jax.experimental import pallas as pl


def kernel(m_indices, m_values, b, d):
    raise NotImplementedError("write your pallas kernel here")


# speedup vs baseline: 137.5745x; 137.5745x over previous
import functools

import jax
import jax.numpy as jnp
from jax import lax
from jax.experimental import pallas as pl
from jax.experimental.pallas import tpu as pltpu
from jax.experimental.pallas import tpu_sc as plsc

N = 100000
E = 6400000
DEGREE = 8

NS = 16
NC = 2
NW = NC * NS
N_PAD = 100352
SLICE = N_PAD // NS
SUB = SLICE // 4
HALF = SLICE // 2

ROW_W = 128
ROW_N = 16
CHUNK = ROW_N * ROW_W
EROWS = E // ROW_W
NBLK = EROWS // ROW_N
BLK_FULL = NBLK // NW
BLK_REM = NBLK % NW

f32 = jnp.float32
i32 = jnp.int32


@functools.lru_cache(maxsize=1)
def _build():
    mesh = plsc.VectorSubcoreMesh(
        core_axis_name="c", subcore_axis_name="s", num_cores=NC,
        num_subcores=NS)

    def _wid():
        c = lax.axis_index("c")
        s = lax.axis_index("s")
        return c, s, c * NS + s

    @functools.partial(
        pl.kernel,
        out_type=(
            jax.ShapeDtypeStruct((EROWS, ROW_W), f32),
            jax.ShapeDtypeStruct((N_PAD,), f32),
            jax.ShapeDtypeStruct((N_PAD,), f32),
            jax.ShapeDtypeStruct((N_PAD,), f32),
        ),
        mesh=mesh,
        compiler_params=pltpu.CompilerParams(needs_layout_passes=False),
        scratch_types=(
            pltpu.VMEM((N_PAD,), f32),
            pltpu.VMEM((ROW_N, ROW_W), i32),
            pltpu.VMEM((ROW_N, ROW_W), i32),
            pltpu.VMEM((ROW_N, ROW_W), f32),
            pltpu.VMEM((ROW_N, ROW_W), f32),
            pltpu.VMEM((SUB,), f32),
            pltpu.VMEM((SUB,), f32),
            pltpu.VMEM((SUB,), f32),
        ),
    )
    def _prep(row_h, col_h, mval_h, b_h, d_h, vals_h, bias_h, pa_h, pb_h,
              dvm, rowb, colb, mvb, outb, bb, db, qb):
        c, s, wid = _wid()
        pltpu.sync_copy(d_h, dvm)

        @pl.when(c == 0)
        def _bias():
            for k in range(4):
                off = s * SLICE + k * SUB
                pltpu.sync_copy(b_h.at[pl.ds(off, SUB)], bb)
                pltpu.sync_copy(d_h.at[pl.ds(off, SUB)], db)

                @pl.loop(0, SUB // 16)
                def _div(j):
                    sl = pl.ds(j * 16, 16)
                    q = bb[sl] / db[sl]
                    qb[sl] = q
                    db[sl] = q * 0.0
                pltpu.sync_copy(qb, bias_h.at[pl.ds(off, SUB)])
                pltpu.sync_copy(qb, pa_h.at[pl.ds(off, SUB)])
                pltpu.sync_copy(db, pb_h.at[pl.ds(off, SUB)])

        nblk = BLK_FULL + jnp.where(wid < BLK_REM, 1, 0)

        @pl.loop(0, nblk)
        def _chunk(g):
            rb = (wid + g * NW) * ROW_N
            pltpu.sync_copy(row_h.at[pl.ds(rb, ROW_N), :], rowb)
            pltpu.sync_copy(col_h.at[pl.ds(rb, ROW_N), :], colb)
            pltpu.sync_copy(mval_h.at[pl.ds(rb, ROW_N), :], mvb)

            @pl.loop(0, ROW_N)
            def _row(r_):
                for o in range(ROW_W // 16):
                    sl = pl.ds(o * 16, 16)
                    r16 = rowb[r_, sl]
                    c16 = colb[r_, sl]
                    dr = plsc.load_gather(dvm, [r16])
                    val = mvb[r_, sl] / dr
                    outb[r_, sl] = jnp.where(r16 == c16, jnp.float32(0.0),
                                             val)
            pltpu.sync_copy(outb, vals_h.at[pl.ds(rb, ROW_N), :])

    @functools.partial(
        pl.kernel,
        out_type=(
            jax.ShapeDtypeStruct((N_PAD,), f32),
            jax.ShapeDtypeStruct((N_PAD,), f32),
            jax.ShapeDtypeStruct((N_PAD,), f32),
        ),
        mesh=mesh,
        compiler_params=pltpu.CompilerParams(needs_layout_passes=False),
        scratch_types=(
            pltpu.VMEM((N_PAD,), f32),
            pltpu.VMEM((ROW_N, ROW_W), i32),
            pltpu.VMEM((ROW_N, ROW_W), i32),
            pltpu.VMEM((ROW_N, ROW_W), f32),
            pltpu.VMEM((ROW_N, ROW_W), f32),
            pltpu.VMEM((SUB,), f32),
            pltpu.VMEM((SUB,), f32),
            pltpu.VMEM((SUB,), f32),
            pltpu.VMEM((SUB,), f32),
            pltpu.VMEM_SHARED((N_PAD,), f32),
            pltpu.VMEM_SHARED((N_PAD,), f32),
        ),
    )
    def _spmv(row_h, col_h, val_h, bias_h, pa_h, pb_h, vcol_h, qa_h, qb_h,
              vvm, rowb, colb, valb, msgb, p0b, p1b, vvb, sdb, v_sh, agg_sh):
        c, s, wid = _wid()
        gate = jnp.where(c == 0, jnp.float32(1.0), jnp.float32(0.0))

        for k in range(4):
            off = s * SLICE + k * SUB
            pltpu.sync_copy(pa_h.at[pl.ds(off, SUB)], p0b)
            pltpu.sync_copy(pb_h.at[pl.ds(off, SUB)], p1b)
            pltpu.sync_copy(bias_h.at[pl.ds(off, SUB)], sdb)

            @pl.loop(0, SUB // 16)
            def _acc(j):
                sl = pl.ds(j * 16, 16)
                vvb[sl] = p0b[sl] + p1b[sl]
                sdb[sl] = sdb[sl] * gate
            pltpu.sync_copy(vvb, v_sh.at[pl.ds(off, SUB)])
            pltpu.sync_copy(sdb, agg_sh.at[pl.ds(off, SUB)])

            @pl.when(c == 0)
            def _wcol():
                pltpu.sync_copy(vvb, vcol_h.at[pl.ds(off, SUB)])

        plsc.subcore_barrier()
        pltpu.sync_copy(v_sh, vvm)

        nblk = BLK_FULL + jnp.where(wid < BLK_REM, 1, 0)

        @pl.loop(0, nblk)
        def _chunk(g):
            rb = (wid + g * NW) * ROW_N
            pltpu.sync_copy(row_h.at[pl.ds(rb, ROW_N), :], rowb)
            pltpu.sync_copy(col_h.at[pl.ds(rb, ROW_N), :], colb)
            pltpu.sync_copy(val_h.at[pl.ds(rb, ROW_N), :], valb)

            @pl.loop(0, ROW_N)
            def _row(r_):
                for o in range(ROW_W // 16):
                    sl = pl.ds(o * 16, 16)
                    idx = colb[r_, sl]
                    x = plsc.load_gather(vvm, [idx])
                    msgb[r_, sl] = x * valb[r_, sl]
                pltpu.sync_copy(msgb.at[r_], agg_sh.at[rowb.at[r_]],
                                add=True)

        plsc.subcore_barrier()

        @pl.when(c == 0)
        def _outa():
            pltpu.sync_copy(agg_sh.at[pl.ds(s * SLICE, SLICE)],
                            qa_h.at[pl.ds(s * SLICE, SLICE)])

        @pl.when(c == 1)
        def _outb():
            pltpu.sync_copy(agg_sh.at[pl.ds(s * SLICE, SLICE)],
                            qb_h.at[pl.ds(s * SLICE, SLICE)])

    CBLEN = (DEGREE + 1) * SLICE

    @functools.partial(
        pl.kernel,
        out_type=jax.ShapeDtypeStruct((N_PAD * (DEGREE + 1),), f32),
        mesh=mesh,
        compiler_params=pltpu.CompilerParams(needs_layout_passes=False),
        scratch_types=(
            pltpu.VMEM((CBLEN,), f32),
            pltpu.VMEM((SLICE,), f32),
            pltpu.VMEM((144,), f32),
            pltpu.VMEM((16,), f32),
            pltpu.VMEM((NS * 16,), f32),
            pltpu.VMEM_SHARED((NS * 16,), f32),
        ),
    )
    def _norm(c0, c1, c2, c3, c4, c5, c6, c7, pa_h, pb_h, out_h,
              cb, tb, ob, mxrow, mxall, mx_sh):
        c, s, _ = _wid()
        off = s * SLICE
        cols = (c0, c1, c2, c3, c4, c5, c6, c7)
        for k in range(DEGREE):
            pltpu.sync_copy(cols[k].at[pl.ds(off, SLICE)],
                            cb.at[pl.ds(k * SLICE, SLICE)])
        pltpu.sync_copy(pa_h.at[pl.ds(off, SLICE)],
                        cb.at[pl.ds(DEGREE * SLICE, SLICE)])
        pltpu.sync_copy(pb_h.at[pl.ds(off, SLICE)], tb)

        @pl.loop(0, SLICE // 16)
        def _add8(g):
            sl = pl.ds(DEGREE * SLICE + g * 16, 16)
            cb[sl] = cb[sl] + tb[pl.ds(g * 16, 16)]

        lanes = lax.iota(i32, 16)
        pvec = jnp.zeros((16,), f32)
        for k in range(DEGREE + 1):
            def mbody(g, m, k=k):
                return jnp.maximum(
                    m, jnp.abs(cb[pl.ds(k * SLICE + g * 16, 16)]))
            m = lax.fori_loop(0, SLICE // 16, mbody, jnp.zeros((16,), f32))
            mk = jnp.max(m)
            pvec = jnp.where(lanes == k, mk, pvec)
        mxrow[...] = pvec
        pltpu.sync_copy(mxrow, mx_sh.at[pl.ds(s * 16, 16)])
        plsc.subcore_barrier()
        pltpu.sync_copy(mx_sh, mxall)
        gmax = jnp.zeros((16,), f32)
        for r in range(NS):
            gmax = jnp.maximum(gmax, mxall[pl.ds(r * 16, 16)])
        ivec = jnp.float32(1.0) / jnp.maximum(gmax, jnp.float32(1e-12))
        inv = []
        for k in range(DEGREE + 1):
            inv.append(jnp.max(jnp.where(lanes == k, ivec, jnp.float32(0.0))))

        base9 = lanes * (DEGREE + 1)

        @pl.loop(0, HALF // 16)
        def _write(g):
            base = c * HALF + g * 16
            for k in range(DEGREE + 1):
                y = cb[pl.ds(k * SLICE + base, 16)] * inv[k]
                plsc.store_scatter(ob, [base9 + k], y)
            pltpu.sync_copy(
                ob, out_h.at[pl.ds((off + base) * (DEGREE + 1), 144)])

    return _prep, _spmv, _norm


def kernel(m_indices, m_values, b, d):
    row = m_indices[0].reshape(EROWS, ROW_W)
    col = m_indices[1].reshape(EROWS, ROW_W)
    mval = m_values.reshape(EROWS, ROW_W)
    b_pad = jnp.pad(b, (0, N_PAD - N))
    d_pad = jnp.pad(d, (0, N_PAD - N), constant_values=1.0)

    _prep, _spmv, _norm = _build()
    vals, bias, pa, pb = _prep(row, col, mval, b_pad, d_pad)
    cols = []
    for _ in range(DEGREE):
        vcol, pa, pb = _spmv(row, col, vals, bias, pa, pb)
        cols.append(vcol)
    out_flat = _norm(*cols, pa, pb)
    return out_flat.reshape(N_PAD, DEGREE + 1)[:N]


# speedup vs baseline: 268.2343x; 1.9497x over previous
import functools

import jax
import jax.numpy as jnp
from jax import lax
from jax.experimental import pallas as pl
from jax.experimental.pallas import tpu as pltpu
from jax.experimental.pallas import tpu_sc as plsc

N = 100000
E = 6400000
DEGREE = 8

NS = 16
NC = 2
NW = NC * NS
N_PAD = 100352
SLICE = N_PAD // NS
SUB = SLICE // 4
HALF = SLICE // 2

ROW_W = 128
ROW_N = 16
CHUNK = ROW_N * ROW_W
EROWS = E // ROW_W
NBLK = EROWS // ROW_N
BLK_FULL = NBLK // NW
BLK_REM = NBLK % NW

f32 = jnp.float32
i32 = jnp.int32


@functools.lru_cache(maxsize=1)
def _build():
    mesh = plsc.VectorSubcoreMesh(
        core_axis_name="c", subcore_axis_name="s", num_cores=NC,
        num_subcores=NS)

    def _wid():
        c = lax.axis_index("c")
        s = lax.axis_index("s")
        return c, s, c * NS + s

    @functools.partial(
        pl.kernel,
        out_type=(
            jax.ShapeDtypeStruct((EROWS, ROW_W), f32),
            jax.ShapeDtypeStruct((N_PAD,), f32),
            jax.ShapeDtypeStruct((N_PAD,), f32),
            jax.ShapeDtypeStruct((N_PAD,), f32),
        ),
        mesh=mesh,
        compiler_params=pltpu.CompilerParams(needs_layout_passes=False),
        scratch_types=(
            pltpu.VMEM((N_PAD,), f32),
            pltpu.VMEM((ROW_N, ROW_W), i32),
            pltpu.VMEM((ROW_N, ROW_W), i32),
            pltpu.VMEM((ROW_N, ROW_W), f32),
            pltpu.VMEM((ROW_N, ROW_W), f32),
            pltpu.VMEM((SUB,), f32),
            pltpu.VMEM((SUB,), f32),
            pltpu.VMEM((SUB,), f32),
        ),
    )
    def _prep(row_h, col_h, mval_h, b_h, d_h, vals_h, bias_h, pa_h, pb_h,
              dvm, rowb, colb, mvb, outb, bb, db, qb):
        c, s, wid = _wid()
        pltpu.sync_copy(d_h, dvm)

        @pl.when(c == 0)
        def _bias():
            for k in range(4):
                off = s * SLICE + k * SUB
                pltpu.sync_copy(b_h.at[pl.ds(off, SUB)], bb)
                pltpu.sync_copy(d_h.at[pl.ds(off, SUB)], db)

                @pl.loop(0, SUB // 16)
                def _div(j):
                    sl = pl.ds(j * 16, 16)
                    q = bb[sl] / db[sl]
                    qb[sl] = q
                    db[sl] = q * 0.0
                pltpu.sync_copy(qb, bias_h.at[pl.ds(off, SUB)])
                pltpu.sync_copy(qb, pa_h.at[pl.ds(off, SUB)])
                pltpu.sync_copy(db, pb_h.at[pl.ds(off, SUB)])

        nblk = BLK_FULL + jnp.where(wid < BLK_REM, 1, 0)

        @pl.loop(0, nblk)
        def _chunk(g):
            rb = (wid + g * NW) * ROW_N
            pltpu.sync_copy(row_h.at[pl.ds(rb, ROW_N), :], rowb)
            pltpu.sync_copy(col_h.at[pl.ds(rb, ROW_N), :], colb)
            pltpu.sync_copy(mval_h.at[pl.ds(rb, ROW_N), :], mvb)

            @pl.loop(0, ROW_N)
            def _row(r_):
                for o in range(ROW_W // 16):
                    sl = pl.ds(o * 16, 16)
                    r16 = rowb[r_, sl]
                    c16 = colb[r_, sl]
                    dr = plsc.load_gather(dvm, [r16])
                    val = mvb[r_, sl] / dr
                    outb[r_, sl] = jnp.where(r16 == c16, jnp.float32(0.0),
                                             val)
            pltpu.sync_copy(outb, vals_h.at[pl.ds(rb, ROW_N), :])

    @functools.partial(
        pl.kernel,
        out_type=(
            jax.ShapeDtypeStruct((N_PAD,), f32),
            jax.ShapeDtypeStruct((N_PAD,), f32),
            jax.ShapeDtypeStruct((N_PAD,), f32),
        ),
        mesh=mesh,
        compiler_params=pltpu.CompilerParams(needs_layout_passes=False),
        scratch_types=(
            pltpu.VMEM((N_PAD,), f32),
            pltpu.VMEM((2, ROW_N, ROW_W), i32),
            pltpu.VMEM((2, ROW_N, ROW_W), i32),
            pltpu.VMEM((2, ROW_N, ROW_W), f32),
            pltpu.VMEM((2, ROW_N, ROW_W), f32),
            pltpu.VMEM((SUB,), f32),
            pltpu.VMEM((SUB,), f32),
            pltpu.VMEM((SUB,), f32),
            pltpu.VMEM((SUB,), f32),
            pltpu.VMEM_SHARED((N_PAD,), f32),
            pltpu.SemaphoreType.DMA,
            pltpu.SemaphoreType.DMA,
            pltpu.SemaphoreType.DMA,
            pltpu.SemaphoreType.DMA,
        ),
    )
    def _spmv(row_h, col_h, val_h, bias_h, pa_h, pb_h, vcol_h, qa_h, qb_h,
              vvm, rowb, colb, valb, msgb, p0b, p1b, vvb, sdb, sh,
              sld0, sld1, ssc0, ssc1):
        c, s, wid = _wid()
        gate = jnp.where(c == 0, jnp.float32(1.0), jnp.float32(0.0))

        for k in range(4):
            off = s * SLICE + k * SUB
            pltpu.sync_copy(pa_h.at[pl.ds(off, SUB)], p0b)
            pltpu.sync_copy(pb_h.at[pl.ds(off, SUB)], p1b)

            @pl.loop(0, SUB // 16)
            def _acc(j):
                sl = pl.ds(j * 16, 16)
                vvb[sl] = p0b[sl] + p1b[sl]
            pltpu.sync_copy(vvb, sh.at[pl.ds(off, SUB)])

            @pl.when(c == 0)
            def _wcol():
                pltpu.sync_copy(vvb, vcol_h.at[pl.ds(off, SUB)])

        plsc.subcore_barrier()
        pltpu.sync_copy(sh, vvm)
        plsc.subcore_barrier()

        for k in range(4):
            off = s * SLICE + k * SUB
            pltpu.sync_copy(bias_h.at[pl.ds(off, SUB)], sdb)

            @pl.loop(0, SUB // 16)
            def _seed(j):
                sl = pl.ds(j * 16, 16)
                sdb[sl] = sdb[sl] * gate
            pltpu.sync_copy(sdb, sh.at[pl.ds(off, SUB)])
        plsc.subcore_barrier()

        nblk = BLK_FULL + jnp.where(wid < BLK_REM, 1, 0)
        sld = (sld0, sld1)
        ssc = (ssc0, ssc1)

        def fire_loads(g, p):
            rb = (wid + g * NW) * ROW_N
            pltpu.async_copy(row_h.at[pl.ds(rb, ROW_N), :], rowb.at[p],
                             sld[p])
            pltpu.async_copy(col_h.at[pl.ds(rb, ROW_N), :], colb.at[p],
                             sld[p])
            pltpu.async_copy(val_h.at[pl.ds(rb, ROW_N), :], valb.at[p],
                             sld[p])

        def wait_loads(p):
            pltpu.make_async_copy(row_h.at[pl.ds(0, ROW_N), :], rowb.at[p],
                                  sld[p]).wait()
            pltpu.make_async_copy(col_h.at[pl.ds(0, ROW_N), :], colb.at[p],
                                  sld[p]).wait()
            pltpu.make_async_copy(val_h.at[pl.ds(0, ROW_N), :], valb.at[p],
                                  sld[p]).wait()

        def wait_scats(p):
            @pl.loop(0, ROW_N)
            def _w(j):
                pltpu.make_async_copy(
                    msgb.at[p].at[j], sh.at[rowb.at[p].at[j]],
                    ssc[p]).wait()

        fire_loads(0, 0)

        @pl.loop(0, BLK_FULL + 1, step=2)
        def _chunk2(g0):
            for p in range(2):
                g = g0 + p

                @pl.when(g < nblk)
                def _do(g=g, p=p):
                    wait_loads(p)

                    @pl.loop(0, ROW_N)
                    def _row(r_):
                        for o in range(ROW_W // 16):
                            sl = pl.ds(o * 16, 16)
                            idx = colb[p, r_, sl]
                            x = plsc.load_gather(vvm, [idx])
                            msgb[p, r_, sl] = x * valb[p, r_, sl]
                        pltpu.async_copy(msgb.at[p].at[r_],
                                         sh.at[rowb.at[p].at[r_]],
                                         ssc[p], add=True)

                    @pl.when(g + 1 < nblk)
                    def _next():
                        @pl.when(g >= 1)
                        def _drain():
                            wait_scats(1 - p)
                        fire_loads(g + 1, 1 - p)

        wait_scats(0)
        wait_scats(1)
        plsc.subcore_barrier()

        @pl.when(c == 0)
        def _outa():
            pltpu.sync_copy(sh.at[pl.ds(s * SLICE, SLICE)],
                            qa_h.at[pl.ds(s * SLICE, SLICE)])

        @pl.when(c == 1)
        def _outb():
            pltpu.sync_copy(sh.at[pl.ds(s * SLICE, SLICE)],
                            qb_h.at[pl.ds(s * SLICE, SLICE)])

    CBLEN = (DEGREE + 1) * SLICE

    @functools.partial(
        pl.kernel,
        out_type=jax.ShapeDtypeStruct((N_PAD * (DEGREE + 1),), f32),
        mesh=mesh,
        compiler_params=pltpu.CompilerParams(needs_layout_passes=False),
        scratch_types=(
            pltpu.VMEM((CBLEN,), f32),
            pltpu.VMEM((SLICE,), f32),
            pltpu.VMEM((144,), f32),
            pltpu.VMEM((16,), f32),
            pltpu.VMEM((NS * 16,), f32),
            pltpu.VMEM_SHARED((NS * 16,), f32),
        ),
    )
    def _norm(c0, c1, c2, c3, c4, c5, c6, c7, pa_h, pb_h, out_h,
              cb, tb, ob, mxrow, mxall, mx_sh):
        c, s, _ = _wid()
        off = s * SLICE
        cols = (c0, c1, c2, c3, c4, c5, c6, c7)
        for k in range(DEGREE):
            pltpu.sync_copy(cols[k].at[pl.ds(off, SLICE)],
                            cb.at[pl.ds(k * SLICE, SLICE)])
        pltpu.sync_copy(pa_h.at[pl.ds(off, SLICE)],
                        cb.at[pl.ds(DEGREE * SLICE, SLICE)])
        pltpu.sync_copy(pb_h.at[pl.ds(off, SLICE)], tb)

        @pl.loop(0, SLICE // 16)
        def _add8(g):
            sl = pl.ds(DEGREE * SLICE + g * 16, 16)
            cb[sl] = cb[sl] + tb[pl.ds(g * 16, 16)]

        lanes = lax.iota(i32, 16)
        pvec = jnp.zeros((16,), f32)
        for k in range(DEGREE + 1):
            def mbody(g, m, k=k):
                return jnp.maximum(
                    m, jnp.abs(cb[pl.ds(k * SLICE + g * 16, 16)]))
            m = lax.fori_loop(0, SLICE // 16, mbody, jnp.zeros((16,), f32))
            mk = jnp.max(m)
            pvec = jnp.where(lanes == k, mk, pvec)
        mxrow[...] = pvec
        pltpu.sync_copy(mxrow, mx_sh.at[pl.ds(s * 16, 16)])
        plsc.subcore_barrier()
        pltpu.sync_copy(mx_sh, mxall)
        gmax = jnp.zeros((16,), f32)
        for r in range(NS):
            gmax = jnp.maximum(gmax, mxall[pl.ds(r * 16, 16)])
        ivec = jnp.float32(1.0) / jnp.maximum(gmax, jnp.float32(1e-12))
        inv = []
        for k in range(DEGREE + 1):
            inv.append(jnp.max(jnp.where(lanes == k, ivec, jnp.float32(0.0))))

        base9 = lanes * (DEGREE + 1)

        @pl.loop(0, HALF // 16)
        def _write(g):
            base = c * HALF + g * 16
            for k in range(DEGREE + 1):
                y = cb[pl.ds(k * SLICE + base, 16)] * inv[k]
                plsc.store_scatter(ob, [base9 + k], y)
            pltpu.sync_copy(
                ob, out_h.at[pl.ds((off + base) * (DEGREE + 1), 144)])

    return _prep, _spmv, _norm


def kernel(m_indices, m_values, b, d):
    row = m_indices[0].reshape(EROWS, ROW_W)
    col = m_indices[1].reshape(EROWS, ROW_W)
    mval = m_values.reshape(EROWS, ROW_W)
    b_pad = jnp.pad(b, (0, N_PAD - N))
    d_pad = jnp.pad(d, (0, N_PAD - N), constant_values=1.0)

    _prep, _spmv, _norm = _build()
    vals, bias, pa, pb = _prep(row, col, mval, b_pad, d_pad)
    cols = []
    for _ in range(DEGREE):
        vcol, pa, pb = _spmv(row, col, vals, bias, pa, pb)
        cols.append(vcol)
    out_flat = _norm(*cols, pa, pb)
    return out_flat.reshape(N_PAD, DEGREE + 1)[:N]


# speedup vs baseline: 287.6991x; 1.0726x over previous
import functools

import jax
import jax.numpy as jnp
from jax import lax
from jax.experimental import pallas as pl
from jax.experimental.pallas import tpu as pltpu
from jax.experimental.pallas import tpu_sc as plsc

N = 100000
E = 6400000
DEGREE = 8

NS = 16
NC = 2
NW = NC * NS
N_PAD = 100352
SLICE = N_PAD // NS
SUB = SLICE // 4
HALF = SLICE // 2

ROW_W = 128
ROW_N = 16
CHUNK = ROW_N * ROW_W
EROWS = E // ROW_W
NBLK = EROWS // ROW_N
BLK_FULL = NBLK // NW
BLK_REM = NBLK % NW

f32 = jnp.float32
i32 = jnp.int32


@functools.lru_cache(maxsize=1)
def _build():
    mesh = plsc.VectorSubcoreMesh(
        core_axis_name="c", subcore_axis_name="s", num_cores=NC,
        num_subcores=NS)

    def _wid():
        c = lax.axis_index("c")
        s = lax.axis_index("s")
        return c, s, c * NS + s

    @functools.partial(
        pl.kernel,
        out_type=(
            jax.ShapeDtypeStruct((EROWS, ROW_W), f32),
            jax.ShapeDtypeStruct((N_PAD,), f32),
            jax.ShapeDtypeStruct((N_PAD,), f32),
            jax.ShapeDtypeStruct((N_PAD,), f32),
        ),
        mesh=mesh,
        compiler_params=pltpu.CompilerParams(needs_layout_passes=False),
        scratch_types=(
            pltpu.VMEM((N_PAD,), f32),
            pltpu.VMEM((2, ROW_N, ROW_W), i32),
            pltpu.VMEM((2, ROW_N, ROW_W), i32),
            pltpu.VMEM((2, ROW_N, ROW_W), f32),
            pltpu.VMEM((2, ROW_N, ROW_W), f32),
            pltpu.VMEM((SUB,), f32),
            pltpu.VMEM((SUB,), f32),
            pltpu.VMEM((SUB,), f32),
            pltpu.SemaphoreType.DMA,
            pltpu.SemaphoreType.DMA,
            pltpu.SemaphoreType.DMA,
            pltpu.SemaphoreType.DMA,
        ),
    )
    def _prep(row_h, col_h, mval_h, b_h, d_h, vals_h, bias_h, pa_h, pb_h,
              dvm, rowb, colb, mvb, outb, bb, db, qb,
              sld0, sld1, swr0, swr1):
        c, s, wid = _wid()
        pltpu.sync_copy(d_h, dvm)

        @pl.when(c == 0)
        def _bias():
            for k in range(4):
                off = s * SLICE + k * SUB
                pltpu.sync_copy(b_h.at[pl.ds(off, SUB)], bb)
                pltpu.sync_copy(d_h.at[pl.ds(off, SUB)], db)

                @pl.loop(0, SUB // 16)
                def _div(j):
                    sl = pl.ds(j * 16, 16)
                    q = bb[sl] / db[sl]
                    qb[sl] = q
                    db[sl] = q * 0.0
                pltpu.sync_copy(qb, bias_h.at[pl.ds(off, SUB)])
                pltpu.sync_copy(qb, pa_h.at[pl.ds(off, SUB)])
                pltpu.sync_copy(db, pb_h.at[pl.ds(off, SUB)])

        nblk = BLK_FULL + jnp.where(wid < BLK_REM, 1, 0)
        sld = (sld0, sld1)
        swr = (swr0, swr1)

        def fire_loads(g, p):
            rb = (wid + g * NW) * ROW_N
            pltpu.async_copy(row_h.at[pl.ds(rb, ROW_N), :], rowb.at[p],
                             sld[p])
            pltpu.async_copy(col_h.at[pl.ds(rb, ROW_N), :], colb.at[p],
                             sld[p])
            pltpu.async_copy(mval_h.at[pl.ds(rb, ROW_N), :], mvb.at[p],
                             sld[p])

        def wait_loads(p):
            pltpu.make_async_copy(row_h.at[pl.ds(0, ROW_N), :], rowb.at[p],
                                  sld[p]).wait()
            pltpu.make_async_copy(col_h.at[pl.ds(0, ROW_N), :], colb.at[p],
                                  sld[p]).wait()
            pltpu.make_async_copy(mval_h.at[pl.ds(0, ROW_N), :], mvb.at[p],
                                  sld[p]).wait()

        def wait_write(p):
            pltpu.make_async_copy(outb.at[p],
                                  vals_h.at[pl.ds(0, ROW_N), :],
                                  swr[p]).wait()

        fire_loads(0, 0)

        @pl.loop(0, BLK_FULL + 1, step=2)
        def _chunk2(g0):
            for p in range(2):
                g = g0 + p

                @pl.when(g < nblk)
                def _do(g=g, p=p):
                    wait_loads(p)

                    @pl.loop(0, ROW_N)
                    def _row(r_):
                        for o in range(ROW_W // 16):
                            sl = pl.ds(o * 16, 16)
                            r16 = rowb[p, r_, sl]
                            c16 = colb[p, r_, sl]
                            dr = plsc.load_gather(dvm, [r16])
                            val = mvb[p, r_, sl] / dr
                            outb[p, r_, sl] = jnp.where(
                                r16 == c16, jnp.float32(0.0), val)
                    rb = (wid + g * NW) * ROW_N
                    pltpu.async_copy(outb.at[p],
                                     vals_h.at[pl.ds(rb, ROW_N), :], swr[p])

                    @pl.when(g + 1 < nblk)
                    def _next():
                        @pl.when(g >= 1)
                        def _drain():
                            wait_write(1 - p)
                        fire_loads(g + 1, 1 - p)

        wait_write(0)
        wait_write(1)

    @functools.partial(
        pl.kernel,
        out_type=(
            jax.ShapeDtypeStruct((N_PAD,), f32),
            jax.ShapeDtypeStruct((N_PAD,), f32),
            jax.ShapeDtypeStruct((N_PAD,), f32),
        ),
        mesh=mesh,
        compiler_params=pltpu.CompilerParams(needs_layout_passes=False),
        scratch_types=(
            pltpu.VMEM((N_PAD,), f32),
            pltpu.VMEM((2, ROW_N, ROW_W), i32),
            pltpu.VMEM((2, ROW_N, ROW_W), i32),
            pltpu.VMEM((2, ROW_N, ROW_W), f32),
            pltpu.VMEM((2, ROW_N, ROW_W), f32),
            pltpu.VMEM((SUB,), f32),
            pltpu.VMEM((SUB,), f32),
            pltpu.VMEM((SUB,), f32),
            pltpu.VMEM((SUB,), f32),
            pltpu.VMEM_SHARED((N_PAD,), f32),
            pltpu.SemaphoreType.DMA,
            pltpu.SemaphoreType.DMA,
            pltpu.SemaphoreType.DMA,
            pltpu.SemaphoreType.DMA,
        ),
    )
    def _spmv(row_h, col_h, val_h, bias_h, pa_h, pb_h, vcol_h, qa_h, qb_h,
              vvm, rowb, colb, valb, msgb, p0b, p1b, vvb, sdb, sh,
              sld0, sld1, ssc0, ssc1):
        c, s, wid = _wid()
        gate = jnp.where(c == 0, jnp.float32(1.0), jnp.float32(0.0))

        for k in range(4):
            off = s * SLICE + k * SUB
            pltpu.sync_copy(pa_h.at[pl.ds(off, SUB)], p0b)
            pltpu.sync_copy(pb_h.at[pl.ds(off, SUB)], p1b)

            @pl.loop(0, SUB // 16)
            def _acc(j):
                sl = pl.ds(j * 16, 16)
                vvb[sl] = p0b[sl] + p1b[sl]
            pltpu.sync_copy(vvb, sh.at[pl.ds(off, SUB)])

            @pl.when(c == 0)
            def _wcol():
                pltpu.sync_copy(vvb, vcol_h.at[pl.ds(off, SUB)])

        plsc.subcore_barrier()
        pltpu.sync_copy(sh, vvm)
        plsc.subcore_barrier()

        for k in range(4):
            off = s * SLICE + k * SUB
            pltpu.sync_copy(bias_h.at[pl.ds(off, SUB)], sdb)

            @pl.loop(0, SUB // 16)
            def _seed(j):
                sl = pl.ds(j * 16, 16)
                sdb[sl] = sdb[sl] * gate
            pltpu.sync_copy(sdb, sh.at[pl.ds(off, SUB)])
        plsc.subcore_barrier()

        nblk = BLK_FULL + jnp.where(wid < BLK_REM, 1, 0)
        sld = (sld0, sld1)
        ssc = (ssc0, ssc1)

        def fire_loads(g, p):
            rb = (wid + g * NW) * ROW_N
            pltpu.async_copy(row_h.at[pl.ds(rb, ROW_N), :], rowb.at[p],
                             sld[p])
            pltpu.async_copy(col_h.at[pl.ds(rb, ROW_N), :], colb.at[p],
                             sld[p])
            pltpu.async_copy(val_h.at[pl.ds(rb, ROW_N), :], valb.at[p],
                             sld[p])

        def wait_loads(p):
            pltpu.make_async_copy(row_h.at[pl.ds(0, ROW_N), :], rowb.at[p],
                                  sld[p]).wait()
            pltpu.make_async_copy(col_h.at[pl.ds(0, ROW_N), :], colb.at[p],
                                  sld[p]).wait()
            pltpu.make_async_copy(val_h.at[pl.ds(0, ROW_N), :], valb.at[p],
                                  sld[p]).wait()

        def wait_scats(p):
            @pl.loop(0, ROW_N)
            def _w(j):
                pltpu.make_async_copy(
                    msgb.at[p].at[j], sh.at[rowb.at[p].at[j]],
                    ssc[p]).wait()

        fire_loads(0, 0)

        @pl.loop(0, BLK_FULL + 1, step=2)
        def _chunk2(g0):
            for p in range(2):
                g = g0 + p

                @pl.when(g < nblk)
                def _do(g=g, p=p):
                    wait_loads(p)

                    @pl.loop(0, ROW_N)
                    def _row(r_):
                        for o in range(ROW_W // 16):
                            sl = pl.ds(o * 16, 16)
                            idx = colb[p, r_, sl]
                            x = plsc.load_gather(vvm, [idx])
                            msgb[p, r_, sl] = x * valb[p, r_, sl]
                        pltpu.async_copy(msgb.at[p].at[r_],
                                         sh.at[rowb.at[p].at[r_]],
                                         ssc[p], add=True)

                    @pl.when(g + 1 < nblk)
                    def _next():
                        @pl.when(g >= 1)
                        def _drain():
                            wait_scats(1 - p)
                        fire_loads(g + 1, 1 - p)

        wait_scats(0)
        wait_scats(1)
        plsc.subcore_barrier()

        @pl.when(c == 0)
        def _outa():
            pltpu.sync_copy(sh.at[pl.ds(s * SLICE, SLICE)],
                            qa_h.at[pl.ds(s * SLICE, SLICE)])

        @pl.when(c == 1)
        def _outb():
            pltpu.sync_copy(sh.at[pl.ds(s * SLICE, SLICE)],
                            qb_h.at[pl.ds(s * SLICE, SLICE)])

    CBLEN = (DEGREE + 1) * SLICE

    @functools.partial(
        pl.kernel,
        out_type=jax.ShapeDtypeStruct((N_PAD * (DEGREE + 1),), f32),
        mesh=mesh,
        compiler_params=pltpu.CompilerParams(needs_layout_passes=False),
        scratch_types=(
            pltpu.VMEM((CBLEN,), f32),
            pltpu.VMEM((SLICE,), f32),
            pltpu.VMEM((288,), f32),
            pltpu.VMEM((16,), f32),
            pltpu.VMEM((NS * 16,), f32),
            pltpu.VMEM_SHARED((NS * 16,), f32),
            pltpu.SemaphoreType.DMA,
            pltpu.SemaphoreType.DMA,
            pltpu.SemaphoreType.DMA,
        ),
    )
    def _norm(c0, c1, c2, c3, c4, c5, c6, c7, pa_h, pb_h, out_h,
              cb, tb, ob, mxrow, mxall, mx_sh, sst, swr0, swr1):
        c, s, _ = _wid()
        off = s * SLICE
        cols = (c0, c1, c2, c3, c4, c5, c6, c7)
        for k in range(DEGREE):
            pltpu.async_copy(cols[k].at[pl.ds(off, SLICE)],
                             cb.at[pl.ds(k * SLICE, SLICE)], sst)
        pltpu.async_copy(pa_h.at[pl.ds(off, SLICE)],
                         cb.at[pl.ds(DEGREE * SLICE, SLICE)], sst)
        pltpu.async_copy(pb_h.at[pl.ds(off, SLICE)], tb, sst)
        for k in range(DEGREE + 1):
            pltpu.make_async_copy(cols[0].at[pl.ds(off, SLICE)],
                                  cb.at[pl.ds(0, SLICE)], sst).wait()
        pltpu.make_async_copy(pb_h.at[pl.ds(off, SLICE)], tb, sst).wait()

        @pl.loop(0, SLICE // 16)
        def _add8(g):
            sl = pl.ds(DEGREE * SLICE + g * 16, 16)
            cb[sl] = cb[sl] + tb[pl.ds(g * 16, 16)]

        lanes = lax.iota(i32, 16)

        def mbody(g, ms):
            return tuple(
                jnp.maximum(ms[k], jnp.abs(cb[pl.ds(k * SLICE + g * 16, 16)]))
                for k in range(DEGREE + 1))
        ms = lax.fori_loop(0, SLICE // 16, mbody,
                           tuple(jnp.zeros((16,), f32)
                                 for _ in range(DEGREE + 1)))
        pvec = jnp.zeros((16,), f32)
        for k in range(DEGREE + 1):
            pvec = jnp.where(lanes == k, jnp.max(ms[k]), pvec)
        mxrow[...] = pvec
        pltpu.sync_copy(mxrow, mx_sh.at[pl.ds(s * 16, 16)])
        plsc.subcore_barrier()
        pltpu.sync_copy(mx_sh, mxall)
        gmax = jnp.zeros((16,), f32)
        for r in range(NS):
            gmax = jnp.maximum(gmax, mxall[pl.ds(r * 16, 16)])
        ivec = jnp.float32(1.0) / jnp.maximum(gmax, jnp.float32(1e-12))
        inv = []
        for k in range(DEGREE + 1):
            inv.append(jnp.max(jnp.where(lanes == k, ivec, jnp.float32(0.0))))

        base9 = lanes * (DEGREE + 1)
        swr = (swr0, swr1)

        def wait_write(p):
            pltpu.make_async_copy(
                ob.at[pl.ds(p * 144, 144)], out_h.at[pl.ds(0, 144)],
                swr[p]).wait()

        @pl.loop(0, HALF // 16, step=2)
        def _write2(g0):
            for p in range(2):
                g = g0 + p

                @pl.when(g >= 2)
                def _drain(p=p):
                    wait_write(p)
                base = c * HALF + g * 16
                for k in range(DEGREE + 1):
                    y = cb[pl.ds(k * SLICE + base, 16)] * inv[k]
                    plsc.store_scatter(ob, [base9 + (k + p * 144)], y)
                pltpu.async_copy(
                    ob.at[pl.ds(p * 144, 144)],
                    out_h.at[pl.ds((off + base) * (DEGREE + 1), 144)],
                    swr[p])

        wait_write(0)
        wait_write(1)

    return _prep, _spmv, _norm


def kernel(m_indices, m_values, b, d):
    row = m_indices[0].reshape(EROWS, ROW_W)
    col = m_indices[1].reshape(EROWS, ROW_W)
    mval = m_values.reshape(EROWS, ROW_W)
    b_pad = jnp.pad(b, (0, N_PAD - N))
    d_pad = jnp.pad(d, (0, N_PAD - N), constant_values=1.0)

    _prep, _spmv, _norm = _build()
    vals, bias, pa, pb = _prep(row, col, mval, b_pad, d_pad)
    cols = []
    for _ in range(DEGREE):
        vcol, pa, pb = _spmv(row, col, vals, bias, pa, pb)
        cols.append(vcol)
    out_flat = _norm(*cols, pa, pb)
    return out_flat.reshape(N_PAD, DEGREE + 1)[:N]
